# Initial kernel scaffold; baseline (speedup 1.0000x reference)
import jax
import jax.numpy as jnp
from jax.experimental import pallas as pl


def kernel(x, edge_index, edge_attr, node_W0, node_b0, node_W1, node_b1, edge_W0, edge_b0, edge_W1, edge_b1, c0_eW, c0_eb, c0_pW, c0_pb, c0_oW, c0_ob, bn0_g, bn0_b, c1_eW, c1_eb, c1_pW, c1_pb, c1_oW, c1_ob, bn1_g, bn1_b, c2_eW, c2_eb, c2_pW, c2_pb, c2_oW, c2_ob, bn2_g, bn2_b, cls_W0, cls_b0, cls_W1, cls_b1, reg_W0, reg_b0, reg_W1, reg_b1):
    raise NotImplementedError("write your pallas kernel here")


# speedup vs baseline: 3.2942x; 3.2942x over previous
import functools

import jax
import jax.numpy as jnp
from jax import lax
from jax.experimental import pallas as pl
from jax.experimental.pallas import tpu as pltpu
from jax.experimental.pallas import tpu_sc as plsc

_N = 10000
_E = 320000
_D = 128
_NCLS = 8
_NBOX = 7

_NW = 32
_EPT = _E // _NW
_KB = 80
_BM = 2000
_BE = 4000
_NBLK_N = _N // _BM


def _full(shape):
    return pl.BlockSpec(shape, lambda i: (0,) * len(shape))


def _rows(bm, ncol):
    return pl.BlockSpec((bm, ncol), lambda i: (i, 0))


def _node_body(x_ref, W0, b0, W1, b1, pWa, pWb, pb, h_ref, a_ref, bq_ref):
    h = jnp.maximum(x_ref[...] @ W0[...] + b0[...], 0.0) @ W1[...] + b1[...]
    h_ref[...] = h
    a_ref[...] = h @ pWa[...] + pb[...]
    bq_ref[...] = h @ pWb[...]


def _node_mlp(x, W0, b0, W1, b1, pWa, pWb, pb):
    ws = (W0, b0, W1, b1, pWa, pWb, pb)
    return pl.pallas_call(
        _node_body,
        grid=(_NBLK_N,),
        in_specs=[_rows(_BM, _D)] + [_full(w.shape) for w in ws],
        out_specs=[_rows(_BM, _D)] * 3,
        out_shape=[jax.ShapeDtypeStruct((_N, _D), jnp.float32)] * 3,
    )(x, *ws)


def _edge_body(ea_ref, eW0, eb0, eW1, eb1,
               e0W, e0b, p0c, e1W, e1b, p1c, e2W, e2b, p2c,
               c0_ref, c1_ref, c2_ref):
    ea2 = jnp.maximum(ea_ref[...] @ eW0[...] + eb0[...], 0.0) @ eW1[...] + eb1[...]
    c0_ref[...] = jnp.maximum(ea2 @ e0W[...] + e0b[...], 0.0) @ p0c[...]
    c1_ref[...] = jnp.maximum(ea2 @ e1W[...] + e1b[...], 0.0) @ p1c[...]
    c2_ref[...] = jnp.maximum(ea2 @ e2W[...] + e2b[...], 0.0) @ p2c[...]


def _edge_mlp(ea, *ws):
    return pl.pallas_call(
        _edge_body,
        grid=(_E // _BE,),
        in_specs=[_rows(_BE, ea.shape[1])] + [_full(w.shape) for w in ws],
        out_specs=[_rows(_BE, _D)] * 3,
        out_shape=[jax.ShapeDtypeStruct((_E, _D), jnp.float32)] * 3,
    )(ea, *ws)


@functools.cache
def _sc_agg():
    mesh = plsc.VectorSubcoreMesh(core_axis_name="c", subcore_axis_name="s")

    @functools.partial(
        pl.kernel,
        mesh=mesh,
        out_type=jax.ShapeDtypeStruct((2 * _N, _D), jnp.float32),
        scratch_types=[
            pltpu.VMEM_SHARED((_N, _D), jnp.float32),
            pltpu.VMEM((_KB,), jnp.int32),
            pltpu.VMEM((_KB,), jnp.int32),
            pltpu.VMEM((_KB, _D), jnp.float32),
            pltpu.VMEM((_KB, _D), jnp.float32),
            pltpu.VMEM((_KB, _D), jnp.float32),
            pltpu.SemaphoreType.DMA,
        ],
    )
    def body(a_h, b_h, c_h, src_h, dst_h, z_h, out_h,
             acc, srcv, dstv, bufA, bufB, bufC, sem):
        cid = lax.axis_index("c")
        sid = lax.axis_index("s")
        wid = cid * 16 + sid
        r0 = sid * 640

        @pl.when(sid < 15)
        def _():
            pltpu.sync_copy(z_h.at[pl.ds(r0, 640)], acc.at[pl.ds(r0, 640)])

        @pl.when(sid == 15)
        def _():
            pltpu.sync_copy(z_h.at[pl.ds(9600, 400)], acc.at[pl.ds(9600, 400)])

        plsc.subcore_barrier()

        def blk(bi, carry):
            base = wid * _EPT + bi * _KB
            pltpu.sync_copy(src_h.at[pl.ds(base, _KB)], srcv)
            pltpu.sync_copy(dst_h.at[pl.ds(base, _KB)], dstv)
            ca = pltpu.async_copy(a_h.at[srcv], bufA, sem)
            cb = pltpu.async_copy(b_h.at[dstv], bufB, sem)
            cc = pltpu.async_copy(c_h.at[pl.ds(base, _KB)], bufC, sem)
            ca.wait()
            cb.wait()
            cc.wait()

            def row(r, carry2):
                for j in range(_D // 16):
                    s = pl.ds(j * 16, 16)
                    bufC[r, s] = jnp.maximum(
                        bufA[r, s] + bufB[r, s] + bufC[r, s], 0.0)
                return carry2

            lax.fori_loop(0, _KB, row, 0)
            pltpu.sync_copy(bufC, acc.at[dstv], add=True)
            return carry

        lax.fori_loop(0, _EPT // _KB, blk, 0)
        plsc.subcore_barrier()

        @pl.when(sid < 15)
        def _():
            pltpu.sync_copy(acc.at[pl.ds(r0, 640)],
                            out_h.at[pl.ds(cid * _N + r0, 640)])

        @pl.when(sid == 15)
        def _():
            pltpu.sync_copy(acc.at[pl.ds(9600, 400)],
                            out_h.at[pl.ds(cid * _N + 9600, 400)])

    return body


def _comb_body(h_ref, g0_ref, g1_ref, oWa, oWb, ob, y_ref, st_ref):
    agg = g0_ref[...] + g1_ref[...]
    y = h_ref[...] @ oWa[...] + agg @ oWb[...] + ob[...]
    y_ref[...] = y
    s1 = jnp.sum(y, axis=0, keepdims=True)
    s2 = jnp.sum(y * y, axis=0, keepdims=True)
    st_ref[0] = jnp.concatenate(
        [s1, s2, jnp.zeros((6, _D), jnp.float32)], axis=0)


def _combine(h, aggP, oWa, oWb, ob):
    ws = (oWa, oWb, ob)
    return pl.pallas_call(
        _comb_body,
        grid=(_NBLK_N,),
        in_specs=[_rows(_BM, _D),
                  pl.BlockSpec((_BM, _D), lambda i: (i, 0)),
                  pl.BlockSpec((_BM, _D), lambda i: (_NBLK_N + i, 0))]
                 + [_full(w.shape) for w in ws],
        out_specs=[_rows(_BM, _D),
                   pl.BlockSpec((1, 8, _D), lambda i: (i, 0, 0))],
        out_shape=[jax.ShapeDtypeStruct((_N, _D), jnp.float32),
                   jax.ShapeDtypeStruct((_NBLK_N, 8, _D), jnp.float32)],
    )(h, aggP, aggP, *ws)


def _bn_from_stats(y, st, g, b):
    mu = jnp.sum(st[:, 0, :], axis=0) * (1.0 / _N)
    m2 = jnp.sum(st[:, 1, :], axis=0) * (1.0 / _N)
    var = m2 - mu * mu
    return jnp.maximum((y - mu) * lax.rsqrt(var + 1e-5) * g + b, 0.0)


def _bnnext_body(y_ref, st_ref, g_ref, b_ref, pWa, pWb, pb,
                 h_ref, a_ref, bq_ref):
    hn = _bn_from_stats(y_ref[...], st_ref[...], g_ref[...], b_ref[...])
    h_ref[...] = hn
    a_ref[...] = hn @ pWa[...] + pb[...]
    bq_ref[...] = hn @ pWb[...]


def _bn_next(y, st, g, b, pWa, pWb, pb):
    ws = (st, g, b, pWa, pWb, pb)
    return pl.pallas_call(
        _bnnext_body,
        grid=(_NBLK_N,),
        in_specs=[_rows(_BM, _D)] + [_full(w.shape) for w in ws],
        out_specs=[_rows(_BM, _D)] * 3,
        out_shape=[jax.ShapeDtypeStruct((_N, _D), jnp.float32)] * 3,
    )(y, *ws)


def _bnheads_body(y_ref, st_ref, g_ref, b_ref,
                  cW0, cb0, cW1, cb1, rW0, rb0, rW1, rb1, c_ref, bb_ref):
    hn = _bn_from_stats(y_ref[...], st_ref[...], g_ref[...], b_ref[...])
    c_ref[...] = jnp.maximum(hn @ cW0[...] + cb0[...], 0.0) @ cW1[...] + cb1[...]
    bb_ref[...] = jnp.maximum(hn @ rW0[...] + rb0[...], 0.0) @ rW1[...] + rb1[...]


def _bn_heads(y, st, g, b, cW0, cb0, cW1, cb1, rW0, rb0, rW1, rb1):
    ws = (st, g, b, cW0, cb0, cW1, cb1, rW0, rb0, rW1, rb1)
    return pl.pallas_call(
        _bnheads_body,
        grid=(_NBLK_N,),
        in_specs=[_rows(_BM, _D)] + [_full(w.shape) for w in ws],
        out_specs=[_rows(_BM, _NCLS), _rows(_BM, _NBOX)],
        out_shape=[jax.ShapeDtypeStruct((_N, _NCLS), jnp.float32),
                   jax.ShapeDtypeStruct((_N, _NBOX), jnp.float32)],
    )(y, *ws)


def kernel(x, edge_index, edge_attr, node_W0, node_b0, node_W1, node_b1,
           edge_W0, edge_b0, edge_W1, edge_b1,
           c0_eW, c0_eb, c0_pW, c0_pb, c0_oW, c0_ob, bn0_g, bn0_b,
           c1_eW, c1_eb, c1_pW, c1_pb, c1_oW, c1_ob, bn1_g, bn1_b,
           c2_eW, c2_eb, c2_pW, c2_pb, c2_oW, c2_ob, bn2_g, bn2_b,
           cls_W0, cls_b0, cls_W1, cls_b1, reg_W0, reg_b0, reg_W1, reg_b1):
    r1 = lambda v: v.reshape(1, -1)
    src = edge_index[0]
    dst = edge_index[1]
    zeros = jnp.zeros((_N, _D), jnp.float32)

    pWs = (c0_pW, c1_pW, c2_pW)
    pbs = (c0_pb, c1_pb, c2_pb)
    oWs = (c0_oW, c1_oW, c2_oW)
    obs = (c0_ob, c1_ob, c2_ob)
    gs = (bn0_g, bn1_g, bn2_g)
    bs = (bn0_b, bn1_b, bn2_b)

    h, A, B = _node_mlp(x, node_W0, r1(node_b0), node_W1, r1(node_b1),
                        c0_pW[:_D], c0_pW[_D:2 * _D], r1(c0_pb))
    Ces = _edge_mlp(edge_attr, edge_W0, r1(edge_b0), edge_W1, r1(edge_b1),
                    c0_eW, r1(c0_eb), c0_pW[2 * _D:],
                    c1_eW, r1(c1_eb), c1_pW[2 * _D:],
                    c2_eW, r1(c2_eb), c2_pW[2 * _D:])

    sc = _sc_agg()
    out_c = out_bb = None
    for l in range(3):
        aggP = sc(A, B, Ces[l], src, dst, zeros)
        y, st = _combine(h, aggP, oWs[l][:_D], oWs[l][_D:], r1(obs[l]))
        if l < 2:
            h, A, B = _bn_next(y, st, r1(gs[l]), r1(bs[l]),
                               pWs[l + 1][:_D], pWs[l + 1][_D:2 * _D],
                               r1(pbs[l + 1]))
        else:
            out_c, out_bb = _bn_heads(y, st, r1(gs[l]), r1(bs[l]),
                                      cls_W0, r1(cls_b0), cls_W1, r1(cls_b1),
                                      reg_W0, r1(reg_b0), reg_W1, r1(reg_b1))
    return (out_c, out_bb)


# speedup vs baseline: 5.4587x; 1.6571x over previous
import functools

import jax
import jax.numpy as jnp
from jax import lax
from jax.experimental import pallas as pl
from jax.experimental.pallas import tpu as pltpu
from jax.experimental.pallas import tpu_sc as plsc

_N = 10000
_E = 320000
_D = 128
_NCLS = 8
_NBOX = 7

_NW = 32
_EPT = _E // _NW
_KB = 40
_BM = 2000
_BE = 4000
_NBLK_N = _N // _BM


def _full(shape):
    return pl.BlockSpec(shape, lambda i: (0,) * len(shape))


def _rows(bm, ncol):
    return pl.BlockSpec((bm, ncol), lambda i: (i, 0))


def _node_body(x_ref, W0, b0, W1, b1, pWa, pWb, pb, h_ref, a_ref, bq_ref):
    h = jnp.maximum(x_ref[...] @ W0[...] + b0[...], 0.0) @ W1[...] + b1[...]
    h_ref[...] = h
    a_ref[...] = h @ pWa[...] + pb[...]
    bq_ref[...] = h @ pWb[...]


def _node_mlp(x, W0, b0, W1, b1, pWa, pWb, pb):
    ws = (W0, b0, W1, b1, pWa, pWb, pb)
    return pl.pallas_call(
        _node_body,
        grid=(_NBLK_N,),
        in_specs=[_rows(_BM, _D)] + [_full(w.shape) for w in ws],
        out_specs=[_rows(_BM, _D)] * 3,
        out_shape=[jax.ShapeDtypeStruct((_N, _D), jnp.float32)] * 3,
    )(x, *ws)


def _edge_body(ea_ref, eW0, eb0, eW1, eb1,
               e0W, e0b, p0c, e1W, e1b, p1c, e2W, e2b, p2c,
               c0_ref, c1_ref, c2_ref):
    ea2 = jnp.maximum(ea_ref[...] @ eW0[...] + eb0[...], 0.0) @ eW1[...] + eb1[...]
    c0_ref[...] = jnp.maximum(ea2 @ e0W[...] + e0b[...], 0.0) @ p0c[...]
    c1_ref[...] = jnp.maximum(ea2 @ e1W[...] + e1b[...], 0.0) @ p1c[...]
    c2_ref[...] = jnp.maximum(ea2 @ e2W[...] + e2b[...], 0.0) @ p2c[...]


def _edge_mlp(ea, *ws):
    return pl.pallas_call(
        _edge_body,
        grid=(_E // _BE,),
        in_specs=[_rows(_BE, ea.shape[1])] + [_full(w.shape) for w in ws],
        out_specs=[_rows(_BE, _D)] * 3,
        out_shape=[jax.ShapeDtypeStruct((_E, _D), jnp.float32)] * 3,
    )(ea, *ws)


_NBLK_E = _EPT // _KB


@functools.cache
def _sc_agg():
    mesh = plsc.VectorSubcoreMesh(core_axis_name="c", subcore_axis_name="s")

    @functools.partial(
        pl.kernel,
        mesh=mesh,
        out_type=jax.ShapeDtypeStruct((2 * _N, _D), jnp.float32),
        scratch_types=[
            pltpu.VMEM_SHARED((_N, _D), jnp.float32),
            pltpu.VMEM((_KB,), jnp.int32),
            pltpu.VMEM((_KB,), jnp.int32),
            pltpu.VMEM((_KB,), jnp.int32),
            pltpu.VMEM((_KB,), jnp.int32),
            pltpu.VMEM((_KB,), jnp.int32),
            pltpu.VMEM((_KB,), jnp.int32),
            pltpu.VMEM((_KB,), jnp.int32),
            pltpu.VMEM((_KB,), jnp.int32),
            pltpu.VMEM((_KB, _D), jnp.float32),
            pltpu.VMEM((_KB, _D), jnp.float32),
            pltpu.VMEM((_KB, _D), jnp.float32),
            pltpu.VMEM((_KB, _D), jnp.float32),
            pltpu.VMEM((_KB, _D), jnp.float32),
            pltpu.VMEM((_KB, _D), jnp.float32),
            pltpu.VMEM((_KB, _D), jnp.float32),
            pltpu.VMEM((_KB, _D), jnp.float32),
            pltpu.SemaphoreType.DMA,
            pltpu.SemaphoreType.DMA,
            pltpu.SemaphoreType.DMA,
            pltpu.SemaphoreType.DMA,
            pltpu.SemaphoreType.DMA,
            pltpu.SemaphoreType.DMA,
        ],
    )
    def body(a_h, b_h, c_h, src_h, dst_h, z_h, out_h,
             acc, sv0, sv1, sv2, sv3, dv0, dv1, dv2, dv3,
             bA0, bA1, bB0, bB1, bC0, bC1, bM0, bM1,
             sg0, sg1, ss0, ss1, si0, si1):
        cid = lax.axis_index("c")
        sid = lax.axis_index("s")
        wid = cid * 16 + sid
        r0 = sid * 640

        @pl.when(sid < 15)
        def _():
            pltpu.sync_copy(z_h.at[pl.ds(r0, 640)], acc.at[pl.ds(r0, 640)])

        @pl.when(sid == 15)
        def _():
            pltpu.sync_copy(z_h.at[pl.ds(9600, 400)], acc.at[pl.ds(9600, 400)])

        plsc.subcore_barrier()

        svs = (sv0, sv1, sv2, sv3)
        dvs = (dv0, dv1, dv2, dv3)
        bAs = (bA0, bA1)
        bBs = (bB0, bB1)
        bCs = (bC0, bC1)
        bMs = (bM0, bM1)
        sgs = (sg0, sg1)
        sss = (ss0, ss1)
        sis = (si0, si1)

        def issue_idx(nbi, q, si):
            pltpu.async_copy(src_h.at[wid, nbi], svs[q], si)
            pltpu.async_copy(dst_h.at[wid, nbi], dvs[q], si)

        def issue_gather(nbi, q, sl):
            base = wid * _EPT + nbi * _KB
            pltpu.async_copy(a_h.at[svs[q]], bAs[sl], sgs[sl])
            pltpu.async_copy(b_h.at[dvs[q]], bBs[sl], sgs[sl])
            pltpu.async_copy(c_h.at[pl.ds(base, _KB)], bCs[sl], sgs[sl])

        def drain_rows(buf, sem):
            pltpu.make_async_copy(a_h.at[pl.ds(0, _KB)], buf, sem).wait()

        def drain_idx(q, si):
            pltpu.make_async_copy(src_h.at[wid, 0], svs[q], si).wait()
            pltpu.make_async_copy(src_h.at[wid, 0], dvs[q], si).wait()

        def compute(sl):
            bA, bB, bC, bM = bAs[sl], bBs[sl], bCs[sl], bMs[sl]

            def row(r, carry):
                for j in range(_D // 16):
                    s = pl.ds(j * 16, 16)
                    bM[r, s] = jnp.maximum(bA[r, s] + bB[r, s] + bC[r, s], 0.0)
                return carry
            lax.fori_loop(0, _KB, row, 0)

        pltpu.sync_copy(src_h.at[wid, 0], sv0)
        pltpu.sync_copy(dst_h.at[wid, 0], dv0)
        pltpu.sync_copy(src_h.at[wid, 1], sv1)
        pltpu.sync_copy(dst_h.at[wid, 1], dv1)
        issue_gather(0, 0, 0)
        issue_gather(1, 1, 1)

        def blk4(i4, carry):
            for u in range(4):
                bi = 4 * i4 + u
                sl = u % 2
                q = u
                q2 = (u + 2) % 4

                @pl.when(bi < _NBLK_E)
                def _():
                    drain_rows(bAs[sl], sgs[sl])
                    drain_rows(bBs[sl], sgs[sl])
                    drain_rows(bCs[sl], sgs[sl])

                    @pl.when(bi >= 2)
                    def _():
                        drain_rows(bMs[sl], sss[sl])

                    @pl.when(bi + 2 < _NBLK_E)
                    def _():
                        issue_idx(bi + 2, q2, sis[sl])

                    compute(sl)
                    pltpu.async_copy(bMs[sl], acc.at[dvs[q]], sss[sl],
                                     add=True)

                    @pl.when(bi + 2 < _NBLK_E)
                    def _():
                        drain_idx(q2, sis[sl])
                        issue_gather(bi + 2, q2, sl)
            return carry

        lax.fori_loop(0, (_NBLK_E + 3) // 4, blk4, 0)
        drain_rows(bM0, ss0)
        drain_rows(bM1, ss1)
        plsc.subcore_barrier()

        @pl.when(sid < 15)
        def _():
            pltpu.sync_copy(acc.at[pl.ds(r0, 640)],
                            out_h.at[pl.ds(cid * _N + r0, 640)])

        @pl.when(sid == 15)
        def _():
            pltpu.sync_copy(acc.at[pl.ds(9600, 400)],
                            out_h.at[pl.ds(cid * _N + 9600, 400)])

    return body


def _comb_body(h_ref, g0_ref, g1_ref, oWa, oWb, ob, y_ref, st_ref):
    agg = g0_ref[...] + g1_ref[...]
    y = h_ref[...] @ oWa[...] + agg @ oWb[...] + ob[...]
    y_ref[...] = y
    s1 = jnp.sum(y, axis=0, keepdims=True)
    s2 = jnp.sum(y * y, axis=0, keepdims=True)
    st_ref[0] = jnp.concatenate(
        [s1, s2, jnp.zeros((6, _D), jnp.float32)], axis=0)


def _combine(h, aggP, oWa, oWb, ob):
    ws = (oWa, oWb, ob)
    return pl.pallas_call(
        _comb_body,
        grid=(_NBLK_N,),
        in_specs=[_rows(_BM, _D),
                  pl.BlockSpec((_BM, _D), lambda i: (i, 0)),
                  pl.BlockSpec((_BM, _D), lambda i: (_NBLK_N + i, 0))]
                 + [_full(w.shape) for w in ws],
        out_specs=[_rows(_BM, _D),
                   pl.BlockSpec((1, 8, _D), lambda i: (i, 0, 0))],
        out_shape=[jax.ShapeDtypeStruct((_N, _D), jnp.float32),
                   jax.ShapeDtypeStruct((_NBLK_N, 8, _D), jnp.float32)],
    )(h, aggP, aggP, *ws)


def _bn_from_stats(y, st, g, b):
    mu = jnp.sum(st[:, 0, :], axis=0) * (1.0 / _N)
    m2 = jnp.sum(st[:, 1, :], axis=0) * (1.0 / _N)
    var = m2 - mu * mu
    return jnp.maximum((y - mu) * lax.rsqrt(var + 1e-5) * g + b, 0.0)


def _bnnext_body(y_ref, st_ref, g_ref, b_ref, pWa, pWb, pb,
                 h_ref, a_ref, bq_ref):
    hn = _bn_from_stats(y_ref[...], st_ref[...], g_ref[...], b_ref[...])
    h_ref[...] = hn
    a_ref[...] = hn @ pWa[...] + pb[...]
    bq_ref[...] = hn @ pWb[...]


def _bn_next(y, st, g, b, pWa, pWb, pb):
    ws = (st, g, b, pWa, pWb, pb)
    return pl.pallas_call(
        _bnnext_body,
        grid=(_NBLK_N,),
        in_specs=[_rows(_BM, _D)] + [_full(w.shape) for w in ws],
        out_specs=[_rows(_BM, _D)] * 3,
        out_shape=[jax.ShapeDtypeStruct((_N, _D), jnp.float32)] * 3,
    )(y, *ws)


def _bnheads_body(y_ref, st_ref, g_ref, b_ref,
                  cW0, cb0, cW1, cb1, rW0, rb0, rW1, rb1, c_ref, bb_ref):
    hn = _bn_from_stats(y_ref[...], st_ref[...], g_ref[...], b_ref[...])
    c_ref[...] = jnp.maximum(hn @ cW0[...] + cb0[...], 0.0) @ cW1[...] + cb1[...]
    bb_ref[...] = jnp.maximum(hn @ rW0[...] + rb0[...], 0.0) @ rW1[...] + rb1[...]


def _bn_heads(y, st, g, b, cW0, cb0, cW1, cb1, rW0, rb0, rW1, rb1):
    ws = (st, g, b, cW0, cb0, cW1, cb1, rW0, rb0, rW1, rb1)
    return pl.pallas_call(
        _bnheads_body,
        grid=(_NBLK_N,),
        in_specs=[_rows(_BM, _D)] + [_full(w.shape) for w in ws],
        out_specs=[_rows(_BM, _NCLS), _rows(_BM, _NBOX)],
        out_shape=[jax.ShapeDtypeStruct((_N, _NCLS), jnp.float32),
                   jax.ShapeDtypeStruct((_N, _NBOX), jnp.float32)],
    )(y, *ws)


def kernel(x, edge_index, edge_attr, node_W0, node_b0, node_W1, node_b1,
           edge_W0, edge_b0, edge_W1, edge_b1,
           c0_eW, c0_eb, c0_pW, c0_pb, c0_oW, c0_ob, bn0_g, bn0_b,
           c1_eW, c1_eb, c1_pW, c1_pb, c1_oW, c1_ob, bn1_g, bn1_b,
           c2_eW, c2_eb, c2_pW, c2_pb, c2_oW, c2_ob, bn2_g, bn2_b,
           cls_W0, cls_b0, cls_W1, cls_b1, reg_W0, reg_b0, reg_W1, reg_b1):
    r1 = lambda v: v.reshape(1, -1)
    src = edge_index[0]
    dst = edge_index[1]
    src3 = src.reshape(_NW, _NBLK_E, _KB)
    dst3 = dst.reshape(_NW, _NBLK_E, _KB)
    zeros = jnp.zeros((_N, _D), jnp.float32)

    pWs = (c0_pW, c1_pW, c2_pW)
    pbs = (c0_pb, c1_pb, c2_pb)
    oWs = (c0_oW, c1_oW, c2_oW)
    obs = (c0_ob, c1_ob, c2_ob)
    gs = (bn0_g, bn1_g, bn2_g)
    bs = (bn0_b, bn1_b, bn2_b)

    h, A, B = _node_mlp(x, node_W0, r1(node_b0), node_W1, r1(node_b1),
                        c0_pW[:_D], c0_pW[_D:2 * _D], r1(c0_pb))
    Ces = _edge_mlp(edge_attr, edge_W0, r1(edge_b0), edge_W1, r1(edge_b1),
                    c0_eW, r1(c0_eb), c0_pW[2 * _D:],
                    c1_eW, r1(c1_eb), c1_pW[2 * _D:],
                    c2_eW, r1(c2_eb), c2_pW[2 * _D:])

    sc = _sc_agg()
    out_c = out_bb = None
    for l in range(3):
        aggP = sc(A, B, Ces[l], src3, dst3, zeros)
        y, st = _combine(h, aggP, oWs[l][:_D], oWs[l][_D:], r1(obs[l]))
        if l < 2:
            h, A, B = _bn_next(y, st, r1(gs[l]), r1(bs[l]),
                               pWs[l + 1][:_D], pWs[l + 1][_D:2 * _D],
                               r1(pbs[l + 1]))
        else:
            out_c, out_bb = _bn_heads(y, st, r1(gs[l]), r1(bs[l]),
                                      cls_W0, r1(cls_b0), cls_W1, r1(cls_b1),
                                      reg_W0, r1(reg_b0), reg_W1, r1(reg_b1))
    return (out_c, out_bb)
